# Initial kernel scaffold; baseline (speedup 1.0000x reference)
import jax
import jax.numpy as jnp
from jax.experimental import pallas as pl


def kernel(input, encoder_weight, glove_weight):
    raise NotImplementedError("write your pallas kernel here")


# speedup vs baseline: 1.1286x; 1.1286x over previous
import functools

import jax
import jax.numpy as jnp
from jax import lax
from jax.experimental import pallas as pl
from jax.experimental.pallas import tpu as pltpu
from jax.experimental.pallas import tpu_sc as plsc

B = 16384
L = 50
D = 32
N = B * L
NTOKEN = 1000000

NC = 2
NS = 16
NW = NC * NS
ROWS_PER_W = N // NW
SUB = 128
CHUNK = 512
NSUB = CHUNK // SUB
NCHUNK = ROWS_PER_W // CHUNK
EPS = 1e-6


def _sqrt16(x):
    xc = jnp.maximum(x, 1e-30)
    i = plsc.bitcast(xc, jnp.int32)
    i = jnp.int32(0x5F3759DF) - lax.shift_right_arithmetic(i, 1)
    y = plsc.bitcast(i, jnp.float32)
    for _ in range(3):
        y = y * (1.5 - 0.5 * xc * y * y)
    return xc * y


def _sc_body(idx_hbm, enc_hbm, glv_hbm, out_e, out_g, part_hbm,
             idx_v, e_v, g_v, acc_v, sem):
    wid = lax.axis_index("s") * NC + lax.axis_index("c")
    eps = jnp.full((16,), EPS, jnp.float32)
    lanes = lax.iota(jnp.int32, 16)

    def chunk_body(c, acc):
        base = wid * ROWS_PER_W + c * CHUNK
        ibase = wid * (ROWS_PER_W // SUB) + c * NSUB
        pltpu.sync_copy(idx_hbm.at[pl.ds(ibase, NSUB)], idx_v)
        handles = []
        for j in range(NSUB):
            dst = pl.ds(j * SUB, SUB)
            handles.append(pltpu.async_copy(enc_hbm.at[idx_v.at[j]], e_v.at[dst], sem))
            handles.append(pltpu.async_copy(glv_hbm.at[idx_v.at[j]], g_v.at[dst], sem))
        for h in handles:
            h.wait()

        def grp_body(g, acc2):
            r0 = g * 16
            x = jnp.zeros((16,), jnp.float32)
            for rr in range(16):
                row = r0 + rr
                d0 = e_v[row, pl.ds(0, 16)] - g_v[row, pl.ds(0, 16)] + eps
                d1 = e_v[row, pl.ds(16, 16)] - g_v[row, pl.ds(16, 16)] + eps
                s = d0 * d0 + d1 * d1
                x = jnp.where(lanes == rr, jnp.sum(s), x)
            return acc2 + _sqrt16(x)

        acc = lax.fori_loop(0, CHUNK // 16, grp_body, acc)
        pltpu.sync_copy(e_v, out_e.at[pl.ds(base, CHUNK)])
        pltpu.sync_copy(g_v, out_g.at[pl.ds(base, CHUNK)])
        return acc

    acc = lax.fori_loop(0, NCHUNK, chunk_body, jnp.zeros((16,), jnp.float32))
    acc_v[...] = acc
    pltpu.sync_copy(acc_v, part_hbm.at[wid])


_sc_call = functools.partial(
    pl.kernel,
    out_type=[
        jax.ShapeDtypeStruct((N, D), jnp.float32),
        jax.ShapeDtypeStruct((N, D), jnp.float32),
        jax.ShapeDtypeStruct((NW, 16), jnp.float32),
    ],
    mesh=plsc.VectorSubcoreMesh(core_axis_name="c", subcore_axis_name="s"),
    compiler_params=pltpu.CompilerParams(
        needs_layout_passes=False, use_tc_tiling_on_sc=False),
    scratch_types=[
        pltpu.VMEM((NSUB, SUB), jnp.int32),
        pltpu.VMEM((CHUNK, D), jnp.float32),
        pltpu.VMEM((CHUNK, D), jnp.float32),
        pltpu.VMEM((16,), jnp.float32),
        pltpu.SemaphoreType.DMA,
    ],
)(_sc_body)


def kernel(input, encoder_weight, glove_weight):
    idx2d = input.astype(jnp.int32).reshape(N // SUB, SUB)
    out_e, out_g, partials = _sc_call(idx2d, encoder_weight, glove_weight)
    emb = out_e.reshape(B, L, D)
    emb_glove = out_g.reshape(B, L, D)
    glove_loss = jnp.sum(partials) / N
    return emb, emb_glove, glove_loss


# speedup vs baseline: 1.1604x; 1.0282x over previous
import functools

import jax
import jax.numpy as jnp
from jax import lax
from jax.experimental import pallas as pl
from jax.experimental.pallas import tpu as pltpu
from jax.experimental.pallas import tpu_sc as plsc

B = 16384
L = 50
D = 32
N = B * L
NTOKEN = 1000000

NC = 2
NS = 16
NW = NC * NS
ROWS_PER_W = N // NW
SUB = 128
CHUNK = 256
NSUB = CHUNK // SUB
NCHUNK = ROWS_PER_W // CHUNK
EPS = 1e-6


def _sqrt16(x):
    xc = jnp.maximum(x, 1e-30)
    i = plsc.bitcast(xc, jnp.int32)
    i = jnp.int32(0x5F3759DF) - lax.shift_right_arithmetic(i, 1)
    y = plsc.bitcast(i, jnp.float32)
    for _ in range(3):
        y = y * (1.5 - 0.5 * xc * y * y)
    return xc * y


def _sc_body(idx_hbm, enc_hbm, glv_hbm, out_e, out_g, part_hbm,
             idx_v, e_v, g_v, acc_v, gsem, wsem):
    wid = lax.axis_index("s") * NC + lax.axis_index("c")
    eps = jnp.full((16,), EPS, jnp.float32)
    lanes = lax.iota(jnp.int32, 16)
    row0 = wid * ROWS_PER_W
    irow0 = wid * (ROWS_PER_W // SUB)

    def load_idx(c, slot):
        pltpu.sync_copy(idx_hbm.at[pl.ds(irow0 + c * NSUB, NSUB)],
                        idx_v.at[slot])

    def fire_gathers(slot):
        for j in range(NSUB):
            dst = pl.ds(j * SUB, SUB)
            pltpu.async_copy(enc_hbm.at[idx_v.at[slot].at[j]],
                             e_v.at[slot].at[dst], gsem.at[slot])
            pltpu.async_copy(glv_hbm.at[idx_v.at[slot].at[j]],
                             g_v.at[slot].at[dst], gsem.at[slot])

    def drain_gathers(slot):
        for j in range(NSUB):
            dst = pl.ds(j * SUB, SUB)
            pltpu.make_async_copy(enc_hbm.at[idx_v.at[slot].at[j]],
                                  e_v.at[slot].at[dst], gsem.at[slot]).wait()
            pltpu.make_async_copy(glv_hbm.at[idx_v.at[slot].at[j]],
                                  g_v.at[slot].at[dst], gsem.at[slot]).wait()

    def compute(slot, acc):
        ev = e_v.at[slot]
        gv = g_v.at[slot]

        def grp_body(g, acc2):
            r0 = g * 16
            x = jnp.zeros((16,), jnp.float32)
            for rr in range(16):
                row = r0 + rr
                d0 = ev[row, pl.ds(0, 16)] - gv[row, pl.ds(0, 16)] + eps
                d1 = ev[row, pl.ds(16, 16)] - gv[row, pl.ds(16, 16)] + eps
                s = d0 * d0 + d1 * d1
                x = jnp.where(lanes == rr, jnp.sum(s), x)
            return acc2 + _sqrt16(x)

        return lax.fori_loop(0, CHUNK // 16, grp_body, acc)

    def phase(c, slot, acc):
        base = row0 + c * CHUNK
        drain_gathers(slot)
        we = pltpu.async_copy(e_v.at[slot], out_e.at[pl.ds(base, CHUNK)],
                              wsem.at[slot])
        wg = pltpu.async_copy(g_v.at[slot], out_g.at[pl.ds(base, CHUNK)],
                              wsem.at[slot])
        acc = compute(slot, acc)
        we.wait()
        wg.wait()

        @pl.when(c + 2 < NCHUNK)
        def _refill():
            load_idx(c + 2, slot)
            fire_gathers(slot)

        return acc

    load_idx(0, 0)
    fire_gathers(0)
    load_idx(1, 1)
    fire_gathers(1)

    def pair_body(c2, acc):
        acc = phase(2 * c2, 0, acc)
        acc = phase(2 * c2 + 1, 1, acc)
        return acc

    acc = lax.fori_loop(0, NCHUNK // 2, pair_body,
                        jnp.zeros((16,), jnp.float32))
    acc_v[...] = acc
    pltpu.sync_copy(acc_v, part_hbm.at[wid])


_sc_call = functools.partial(
    pl.kernel,
    out_type=[
        jax.ShapeDtypeStruct((N, D), jnp.float32),
        jax.ShapeDtypeStruct((N, D), jnp.float32),
        jax.ShapeDtypeStruct((NW, 16), jnp.float32),
    ],
    mesh=plsc.VectorSubcoreMesh(core_axis_name="c", subcore_axis_name="s"),
    compiler_params=pltpu.CompilerParams(
        needs_layout_passes=False, use_tc_tiling_on_sc=False),
    scratch_types=[
        pltpu.VMEM((2, NSUB, SUB), jnp.int32),
        pltpu.VMEM((2, CHUNK, D), jnp.float32),
        pltpu.VMEM((2, CHUNK, D), jnp.float32),
        pltpu.VMEM((16,), jnp.float32),
        pltpu.SemaphoreType.DMA((2,)),
        pltpu.SemaphoreType.DMA((2,)),
    ],
)(_sc_body)


def kernel(input, encoder_weight, glove_weight):
    idx2d = input.astype(jnp.int32).reshape(N // SUB, SUB)
    out_e, out_g, partials = _sc_call(idx2d, encoder_weight, glove_weight)
    emb = out_e.reshape(B, L, D)
    emb_glove = out_g.reshape(B, L, D)
    glove_loss = jnp.sum(partials) / N
    return emb, emb_glove, glove_loss


# speedup vs baseline: 1.8784x; 1.6187x over previous
import functools

import jax
import jax.numpy as jnp
from jax import lax
from jax.experimental import pallas as pl
from jax.experimental.pallas import tpu as pltpu
from jax.experimental.pallas import tpu_sc as plsc

B = 16384
L = 50
D = 32
N = B * L
NTOKEN = 1000000

NC = 2
NS = 16
NW = NC * NS
ROWS_PER_W = N // NW
SUB = 128
CHUNK = 256
NSUB = CHUNK // SUB
NCHUNK = ROWS_PER_W // CHUNK
OROWS = CHUNK * D // 128
EPS = 1e-6


def _sqrt16(x):
    xc = jnp.maximum(x, 1e-30)
    i = plsc.bitcast(xc, jnp.int32)
    i = jnp.int32(0x5F3759DF) - lax.shift_right_arithmetic(i, 1)
    y = plsc.bitcast(i, jnp.float32)
    for _ in range(3):
        y = y * (1.5 - 0.5 * xc * y * y)
    return xc * y


def _sc_body(idx_hbm, enc_hbm, glv_hbm, out_e, out_g, part_hbm,
             idx_v, e_v, g_v, ew_v, gw_v, acc_v, gsem, wsem):
    wid = lax.axis_index("s") * NC + lax.axis_index("c")
    eps = jnp.full((16,), EPS, jnp.float32)
    lanes = lax.iota(jnp.int32, 16)
    row0 = wid * ROWS_PER_W
    irow0 = wid * (ROWS_PER_W // SUB)

    def load_idx(c, slot):
        pltpu.sync_copy(idx_hbm.at[pl.ds(irow0 + c * NSUB, NSUB)],
                        idx_v.at[slot])

    def fire_gathers(slot):
        for j in range(NSUB):
            dst = pl.ds(j * SUB, SUB)
            pltpu.async_copy(enc_hbm.at[idx_v.at[slot].at[j]],
                             e_v.at[slot].at[dst], gsem.at[slot])
            pltpu.async_copy(glv_hbm.at[idx_v.at[slot].at[j]],
                             g_v.at[slot].at[dst], gsem.at[slot])

    def drain_gathers(slot):
        for j in range(NSUB):
            dst = pl.ds(j * SUB, SUB)
            pltpu.make_async_copy(enc_hbm.at[idx_v.at[slot].at[j]],
                                  e_v.at[slot].at[dst], gsem.at[slot]).wait()
            pltpu.make_async_copy(glv_hbm.at[idx_v.at[slot].at[j]],
                                  g_v.at[slot].at[dst], gsem.at[slot]).wait()

    def drain_writeback(slot):
        pltpu.make_async_copy(ew_v.at[slot], out_e.at[pl.ds(0, OROWS)],
                              wsem.at[slot]).wait()
        pltpu.make_async_copy(gw_v.at[slot], out_g.at[pl.ds(0, OROWS)],
                              wsem.at[slot]).wait()

    def compute(slot, acc):
        ev = e_v.at[slot]
        gv = g_v.at[slot]
        ew = ew_v.at[slot]
        gw = gw_v.at[slot]

        def grp_body(g, acc2):
            r0 = g * 16
            x = jnp.zeros((16,), jnp.float32)
            for rr in range(16):
                row = r0 + rr
                orow = 4 * g + rr // 4
                ocol = (rr % 4) * 32
                e0 = ev[row, pl.ds(0, 16)]
                e1 = ev[row, pl.ds(16, 16)]
                g0 = gv[row, pl.ds(0, 16)]
                g1 = gv[row, pl.ds(16, 16)]
                ew[orow, pl.ds(ocol, 16)] = e0
                ew[orow, pl.ds(ocol + 16, 16)] = e1
                gw[orow, pl.ds(ocol, 16)] = g0
                gw[orow, pl.ds(ocol + 16, 16)] = g1
                d0 = e0 - g0 + eps
                d1 = e1 - g1 + eps
                s = d0 * d0 + d1 * d1
                x = jnp.where(lanes == rr, jnp.sum(s), x)
            return acc2 + _sqrt16(x)

        return lax.fori_loop(0, CHUNK // 16, grp_body, acc)

    def phase(c, slot, acc, first):
        drain_gathers(slot)
        if not first:
            drain_writeback(slot)
        acc = compute(slot, acc)
        obase = (row0 + c * CHUNK) // 4
        pltpu.async_copy(ew_v.at[slot], out_e.at[pl.ds(obase, OROWS)],
                         wsem.at[slot])
        pltpu.async_copy(gw_v.at[slot], out_g.at[pl.ds(obase, OROWS)],
                         wsem.at[slot])

        @pl.when(c + 2 < NCHUNK)
        def _refill():
            load_idx(c + 2, slot)
            fire_gathers(slot)

        return acc

    load_idx(0, 0)
    fire_gathers(0)
    load_idx(1, 1)
    fire_gathers(1)

    zero = jnp.zeros((16,), jnp.float32)
    acc = phase(0, 0, zero, True)
    acc = phase(1, 1, acc, True)

    def pair_body(c2, acc):
        acc = phase(2 * c2, 0, acc, False)
        acc = phase(2 * c2 + 1, 1, acc, False)
        return acc

    acc = lax.fori_loop(1, NCHUNK // 2, pair_body, acc)
    drain_writeback(0)
    drain_writeback(1)
    acc_v[...] = acc
    pltpu.sync_copy(acc_v, part_hbm.at[wid])


_sc_call = functools.partial(
    pl.kernel,
    out_type=[
        jax.ShapeDtypeStruct((N * D // 128, 128), jnp.float32),
        jax.ShapeDtypeStruct((N * D // 128, 128), jnp.float32),
        jax.ShapeDtypeStruct((NW, 16), jnp.float32),
    ],
    mesh=plsc.VectorSubcoreMesh(core_axis_name="c", subcore_axis_name="s"),
    compiler_params=pltpu.CompilerParams(
        needs_layout_passes=False, use_tc_tiling_on_sc=False),
    scratch_types=[
        pltpu.VMEM((2, NSUB, SUB), jnp.int32),
        pltpu.VMEM((2, CHUNK, D), jnp.float32),
        pltpu.VMEM((2, CHUNK, D), jnp.float32),
        pltpu.VMEM((2, OROWS, 128), jnp.float32),
        pltpu.VMEM((2, OROWS, 128), jnp.float32),
        pltpu.VMEM((16,), jnp.float32),
        pltpu.SemaphoreType.DMA((2,)),
        pltpu.SemaphoreType.DMA((2,)),
    ],
)(_sc_body)


def kernel(input, encoder_weight, glove_weight):
    idx2d = input.astype(jnp.int32).reshape(N // SUB, SUB)
    out_e, out_g, partials = _sc_call(idx2d, encoder_weight, glove_weight)
    emb = out_e.reshape(B, L, D)
    emb_glove = out_g.reshape(B, L, D)
    glove_loss = jnp.sum(partials) / N
    return emb, emb_glove, glove_loss


# speedup vs baseline: 2.3699x; 1.2616x over previous
import functools

import jax
import jax.numpy as jnp
from jax import lax
from jax.experimental import pallas as pl
from jax.experimental.pallas import tpu as pltpu
from jax.experimental.pallas import tpu_sc as plsc

B = 16384
L = 50
D = 32
N = B * L
NTOKEN = 1000000

TBLK = 16384
QUART = TBLK // 4
TGRID = -(-NTOKEN // TBLK)
NTPAD = TGRID * TBLK

NC = 2
NS = 16
NW = NC * NS
ROWS_PER_W = N // NW
SUB = 128
CHUNK = 256
NSUB = CHUNK // SUB
NCHUNK = ROWS_PER_W // CHUNK
OROWS = CHUNK * D // 128
EPS = 1e-6


def _sqrt16(x):
    xc = jnp.maximum(x, 1e-30)
    i = plsc.bitcast(xc, jnp.int32)
    i = jnp.int32(0x5F3759DF) - lax.shift_right_arithmetic(i, 1)
    y = plsc.bitcast(i, jnp.float32)
    for _ in range(3):
        y = y * (1.5 - 0.5 * xc * y * y)
    return xc * y


def _tc_lin_body(e_ref, g_ref, oe_ref, og_ref):
    for src, dst in ((e_ref, oe_ref), (g_ref, og_ref)):
        y = jnp.transpose(src[...], (1, 0))
        dst[...] = jnp.concatenate(
            [y[q * QUART:(q + 1) * QUART] for q in range(4)], axis=1)


_tc_lin = pl.pallas_call(
    _tc_lin_body,
    grid=(TGRID,),
    in_specs=[
        pl.BlockSpec((32, TBLK), lambda i: (0, i)),
        pl.BlockSpec((32, TBLK), lambda i: (0, i)),
    ],
    out_specs=[
        pl.BlockSpec((TBLK * 32 // 128, 128), lambda i: (i, 0)),
        pl.BlockSpec((TBLK * 32 // 128, 128), lambda i: (i, 0)),
    ],
    out_shape=[
        jax.ShapeDtypeStruct((NTPAD * 32 // 128, 128), jnp.float32),
        jax.ShapeDtypeStruct((NTPAD * 32 // 128, 128), jnp.float32),
    ],
)


def _sc_body(idx_hbm, enc_hbm, glv_hbm, out_e, out_g, part_hbm,
             idx_v, e_v, g_v, ew_v, gw_v, acc_v, gsem, wsem):
    wid = lax.axis_index("s") * NC + lax.axis_index("c")
    eps = jnp.full((16,), EPS, jnp.float32)
    lanes = lax.iota(jnp.int32, 16)
    row0 = wid * ROWS_PER_W
    irow0 = wid * (ROWS_PER_W // SUB)

    def load_idx(c, slot):
        pltpu.sync_copy(idx_hbm.at[pl.ds(irow0 + c * NSUB, NSUB)],
                        idx_v.at[slot])
        for j in range(NSUB):
            for k in range(SUB // 16):
                sl = pl.ds(k * 16, 16)
                t = idx_v[slot, j, sl]
                blk = lax.shift_right_logical(t, 14)
                rem = lax.bitwise_and(t, jnp.int32(TBLK - 1))
                q = lax.shift_right_logical(rem, 12)
                r = lax.bitwise_and(rem, jnp.int32(QUART - 1))
                idx_v[slot, j, sl] = (
                    lax.shift_left(blk, 14) + lax.shift_left(r, 2) + q)

    def fire_gathers(slot):
        for j in range(NSUB):
            dst = pl.ds(j * SUB, SUB)
            pltpu.async_copy(enc_hbm.at[idx_v.at[slot].at[j]],
                             e_v.at[slot].at[dst], gsem.at[slot])
            pltpu.async_copy(glv_hbm.at[idx_v.at[slot].at[j]],
                             g_v.at[slot].at[dst], gsem.at[slot])

    def drain_gathers(slot):
        for j in range(NSUB):
            dst = pl.ds(j * SUB, SUB)
            pltpu.make_async_copy(enc_hbm.at[idx_v.at[slot].at[j]],
                                  e_v.at[slot].at[dst], gsem.at[slot]).wait()
            pltpu.make_async_copy(glv_hbm.at[idx_v.at[slot].at[j]],
                                  g_v.at[slot].at[dst], gsem.at[slot]).wait()

    def drain_writeback(slot):
        pltpu.make_async_copy(ew_v.at[slot], out_e.at[pl.ds(0, OROWS)],
                              wsem.at[slot]).wait()
        pltpu.make_async_copy(gw_v.at[slot], out_g.at[pl.ds(0, OROWS)],
                              wsem.at[slot]).wait()

    def compute(slot, acc):
        ev = e_v.at[slot]
        gv = g_v.at[slot]
        ew = ew_v.at[slot]
        gw = gw_v.at[slot]

        def grp_body(g, acc2):
            r0 = g * 16
            x = jnp.zeros((16,), jnp.float32)
            for rr in range(16):
                row = r0 + rr
                orow = 4 * g + rr // 4
                ocol = (rr % 4) * 32
                e0 = ev[row, pl.ds(0, 16)]
                e1 = ev[row, pl.ds(16, 16)]
                g0 = gv[row, pl.ds(0, 16)]
                g1 = gv[row, pl.ds(16, 16)]
                ew[orow, pl.ds(ocol, 16)] = e0
                ew[orow, pl.ds(ocol + 16, 16)] = e1
                gw[orow, pl.ds(ocol, 16)] = g0
                gw[orow, pl.ds(ocol + 16, 16)] = g1
                d0 = e0 - g0 + eps
                d1 = e1 - g1 + eps
                s = d0 * d0 + d1 * d1
                x = jnp.where(lanes == rr, jnp.sum(s), x)
            return acc2 + _sqrt16(x)

        return lax.fori_loop(0, CHUNK // 16, grp_body, acc)

    def phase(c, slot, acc, first):
        drain_gathers(slot)
        if not first:
            drain_writeback(slot)
        acc = compute(slot, acc)
        obase = (row0 + c * CHUNK) // 4
        pltpu.async_copy(ew_v.at[slot], out_e.at[pl.ds(obase, OROWS)],
                         wsem.at[slot])
        pltpu.async_copy(gw_v.at[slot], out_g.at[pl.ds(obase, OROWS)],
                         wsem.at[slot])

        @pl.when(c + 2 < NCHUNK)
        def _refill():
            load_idx(c + 2, slot)
            fire_gathers(slot)

        return acc

    load_idx(0, 0)
    fire_gathers(0)
    load_idx(1, 1)
    fire_gathers(1)

    zero = jnp.zeros((16,), jnp.float32)
    acc = phase(0, 0, zero, True)
    acc = phase(1, 1, acc, True)

    def pair_body(c2, acc):
        acc = phase(2 * c2, 0, acc, False)
        acc = phase(2 * c2 + 1, 1, acc, False)
        return acc

    acc = lax.fori_loop(1, NCHUNK // 2, pair_body, acc)
    drain_writeback(0)
    drain_writeback(1)
    acc_v[...] = acc
    pltpu.sync_copy(acc_v, part_hbm.at[wid])


_sc_call = functools.partial(
    pl.kernel,
    out_type=[
        jax.ShapeDtypeStruct((N * D // 128, 128), jnp.float32),
        jax.ShapeDtypeStruct((N * D // 128, 128), jnp.float32),
        jax.ShapeDtypeStruct((NW, 16), jnp.float32),
    ],
    mesh=plsc.VectorSubcoreMesh(core_axis_name="c", subcore_axis_name="s"),
    compiler_params=pltpu.CompilerParams(
        needs_layout_passes=False, use_tc_tiling_on_sc=False),
    scratch_types=[
        pltpu.VMEM((2, NSUB, SUB), jnp.int32),
        pltpu.VMEM((2, CHUNK, D), jnp.float32),
        pltpu.VMEM((2, CHUNK, D), jnp.float32),
        pltpu.VMEM((2, OROWS, 128), jnp.float32),
        pltpu.VMEM((2, OROWS, 128), jnp.float32),
        pltpu.VMEM((16,), jnp.float32),
        pltpu.SemaphoreType.DMA((2,)),
        pltpu.SemaphoreType.DMA((2,)),
    ],
)(_sc_body)


def kernel(input, encoder_weight, glove_weight):
    idx2d = input.astype(jnp.int32).reshape(N // SUB, SUB)
    enc_lin, glv_lin = _tc_lin(encoder_weight.T, glove_weight.T)
    enc_lin = enc_lin.reshape(NTPAD, D)
    glv_lin = glv_lin.reshape(NTPAD, D)
    out_e, out_g, partials = _sc_call(idx2d, enc_lin, glv_lin)
    emb = out_e.reshape(B, L, D)
    emb_glove = out_g.reshape(B, L, D)
    glove_loss = jnp.sum(partials) / N
    return emb, emb_glove, glove_loss
